# Initial kernel scaffold; baseline (speedup 1.0000x reference)
import jax
import jax.numpy as jnp
from jax.experimental import pallas as pl


def kernel(UserIdx, itemIdx, user_table, item_table):
    raise NotImplementedError("write your pallas kernel here")


# speedup vs baseline: 1.9954x; 1.9954x over previous
import dataclasses

import jax
import jax.numpy as jnp
from jax import lax
from jax.experimental import pallas as pl
from jax.experimental.pallas import tpu as pltpu
from jax.experimental.pallas import tpu_sc as plsc

_B = 16384
_DIM = 128
_NC = 2
_NS = 16
_NW = _NC * _NS
_BPW = _B // _NW
_C = 128
_NCHUNK = _BPW // _C
_L = 16
_VPR = _DIM // _L


def _mf_body(uidx_hbm, iidx_hbm, utab_hbm, itab_hbm, out_hbm,
             uidx_v, iidx_v, urows, irows, out_v):
    wid = lax.axis_index("subcore") * _NC + lax.axis_index("core")
    base = wid * _BPW

    for ch in range(_NCHUNK):
        pltpu.sync_copy(uidx_hbm.at[pl.ds(base + ch * _C, _C)], uidx_v.at[ch])
        pltpu.sync_copy(iidx_hbm.at[pl.ds(base + ch * _C, _C)], iidx_v.at[ch])

    for ch in range(_NCHUNK):
        pltpu.sync_copy(utab_hbm.at[uidx_v.at[ch]], urows)
        pltpu.sync_copy(itab_hbm.at[iidx_v.at[ch]], irows)

        @pl.loop(0, _C // _L)
        def _(g, ch=ch):
            lane = lax.iota(jnp.int32, _L)
            res = jnp.zeros((_L,), jnp.float32)
            for j in range(_L):
                r = g * _L + j
                acc = urows[r, pl.ds(0, _L)] * irows[r, pl.ds(0, _L)]
                for k in range(1, _VPR):
                    acc = acc + (urows[r, pl.ds(k * _L, _L)] *
                                 irows[r, pl.ds(k * _L, _L)])
                res = jnp.where(lane == j, jnp.sum(acc), res)
            out_v[pl.ds(ch * _C + g * _L, _L)] = res

    pltpu.sync_copy(out_v, out_hbm.at[pl.ds(base, _BPW)])


def kernel(UserIdx, itemIdx, user_table, item_table):
    mesh = plsc.VectorSubcoreMesh(core_axis_name="core",
                                  subcore_axis_name="subcore")
    cp = pltpu.CompilerParams()
    if "needs_layout_passes" in pltpu.CompilerParams.__dataclass_fields__:
        cp = dataclasses.replace(cp, needs_layout_passes=False)
    mf = pl.kernel(
        _mf_body,
        out_type=jax.ShapeDtypeStruct((_B,), jnp.float32),
        mesh=mesh,
        scratch_types=[
            pltpu.VMEM((_NCHUNK, _C), jnp.int32),
            pltpu.VMEM((_NCHUNK, _C), jnp.int32),
            pltpu.VMEM((_C, _DIM), jnp.float32),
            pltpu.VMEM((_C, _DIM), jnp.float32),
            pltpu.VMEM((_BPW,), jnp.float32),
        ],
        compiler_params=cp,
    )
    return mf(UserIdx.astype(jnp.int32), itemIdx.astype(jnp.int32),
              user_table, item_table)


# speedup vs baseline: 2.2422x; 1.1237x over previous
import dataclasses

import jax
import jax.numpy as jnp
from jax import lax
from jax.experimental import pallas as pl
from jax.experimental.pallas import tpu as pltpu
from jax.experimental.pallas import tpu_sc as plsc

_B = 16384
_DIM = 128
_NC = 2
_NS = 16
_NW = _NC * _NS
_BPW = _B // _NW
_C = 128
_NCHUNK = _BPW // _C
_L = 16
_VPR = _DIM // _L


def _mf_body(uidx_hbm, iidx_hbm, utab_hbm, itab_hbm, out_hbm,
             uidx_v, iidx_v, urows, irows, out_v,
             sem_u0, sem_u1, sem_i0, sem_i1):
    wid = lax.axis_index("subcore") * _NC + lax.axis_index("core")
    base = wid * _BPW
    sems_u = (sem_u0, sem_u1)
    sems_i = (sem_i0, sem_i1)

    for ch in range(_NCHUNK):
        pltpu.sync_copy(uidx_hbm.at[pl.ds(base + ch * _C, _C)], uidx_v.at[ch])
        pltpu.sync_copy(iidx_hbm.at[pl.ds(base + ch * _C, _C)], iidx_v.at[ch])

    def start(ch, buf):
        cu = pltpu.async_copy(utab_hbm.at[uidx_v.at[ch]], urows.at[buf],
                              sems_u[buf])
        ci = pltpu.async_copy(itab_hbm.at[iidx_v.at[ch]], irows.at[buf],
                              sems_i[buf])
        return cu, ci

    pend = start(0, 0)
    for ch in range(_NCHUNK):
        buf = ch % 2
        cur = pend
        if ch + 1 < _NCHUNK:
            pend = start(ch + 1, 1 - buf)
        cur[0].wait()
        cur[1].wait()

        @pl.loop(0, _C // _L)
        def _(g, ch=ch, buf=buf):
            lane = lax.iota(jnp.int32, _L)
            res = jnp.zeros((_L,), jnp.float32)
            for j in range(_L):
                r = g * _L + j
                acc = urows[buf, r, pl.ds(0, _L)] * irows[buf, r, pl.ds(0, _L)]
                for k in range(1, _VPR):
                    acc = acc + (urows[buf, r, pl.ds(k * _L, _L)] *
                                 irows[buf, r, pl.ds(k * _L, _L)])
                res = jnp.where(lane == j, jnp.sum(acc), res)
            out_v[pl.ds(ch * _C + g * _L, _L)] = res

    pltpu.sync_copy(out_v, out_hbm.at[pl.ds(base, _BPW)])


def kernel(UserIdx, itemIdx, user_table, item_table):
    mesh = plsc.VectorSubcoreMesh(core_axis_name="core",
                                  subcore_axis_name="subcore")
    cp = pltpu.CompilerParams()
    if "needs_layout_passes" in pltpu.CompilerParams.__dataclass_fields__:
        cp = dataclasses.replace(cp, needs_layout_passes=False)
    mf = pl.kernel(
        _mf_body,
        out_type=jax.ShapeDtypeStruct((_B,), jnp.float32),
        mesh=mesh,
        scratch_types=[
            pltpu.VMEM((_NCHUNK, _C), jnp.int32),
            pltpu.VMEM((_NCHUNK, _C), jnp.int32),
            pltpu.VMEM((2, _C, _DIM), jnp.float32),
            pltpu.VMEM((2, _C, _DIM), jnp.float32),
            pltpu.VMEM((_BPW,), jnp.float32),
            pltpu.SemaphoreType.DMA,
            pltpu.SemaphoreType.DMA,
            pltpu.SemaphoreType.DMA,
            pltpu.SemaphoreType.DMA,
        ],
        compiler_params=cp,
    )
    return mf(UserIdx.astype(jnp.int32), itemIdx.astype(jnp.int32),
              user_table, item_table)


# speedup vs baseline: 3.0343x; 1.3532x over previous
import dataclasses

import jax
import jax.numpy as jnp
from jax import lax
from jax.experimental import pallas as pl
from jax.experimental.pallas import tpu as pltpu
from jax.experimental.pallas import tpu_sc as plsc

_B = 16384
_DIM = 128
_NC = 2
_NS = 16
_NW = _NC * _NS
_BPW = _B // _NW
_C = 128
_NCHUNK = _BPW // _C
_L = 16
_VPR = _DIM // _L


_PAD = _L + 1


def _mf_body(uidx_hbm, iidx_hbm, utab_hbm, itab_hbm, out_hbm,
             uidx_v, iidx_v, urows, irows, out_v, part_v,
             sem_u0, sem_u1, sem_i0, sem_i1):
    wid = lax.axis_index("subcore") * _NC + lax.axis_index("core")
    base = wid * _BPW
    sems_u = (sem_u0, sem_u1)
    sems_i = (sem_i0, sem_i1)

    for ch in range(_NCHUNK):
        pltpu.sync_copy(uidx_hbm.at[pl.ds(base + ch * _C, _C)], uidx_v.at[ch])
        pltpu.sync_copy(iidx_hbm.at[pl.ds(base + ch * _C, _C)], iidx_v.at[ch])

    def start(ch, buf):
        cu = pltpu.async_copy(utab_hbm.at[uidx_v.at[ch]], urows.at[buf],
                              sems_u[buf])
        ci = pltpu.async_copy(itab_hbm.at[iidx_v.at[ch]], irows.at[buf],
                              sems_i[buf])
        return cu, ci

    pend = start(0, 0)
    for ch in range(_NCHUNK):
        buf = ch % 2
        cur = pend
        if ch + 1 < _NCHUNK:
            pend = start(ch + 1, 1 - buf)
        cur[0].wait()
        cur[1].wait()

        @pl.loop(0, _C // _L)
        def _(g, ch=ch, buf=buf):
            for j in range(_L):
                r = g * _L + j
                acc = urows[buf, r, pl.ds(0, _L)] * irows[buf, r, pl.ds(0, _L)]
                for k in range(1, _VPR):
                    acc = acc + (urows[buf, r, pl.ds(k * _L, _L)] *
                                 irows[buf, r, pl.ds(k * _L, _L)])
                part_v[pl.ds(j * _PAD, _L)] = acc
            col = lax.iota(jnp.int32, _L) * _PAD
            res = plsc.load_gather(part_v, [col])
            for l in range(1, _L):
                res = res + plsc.load_gather(part_v, [col + l])
            out_v[pl.ds(ch * _C + g * _L, _L)] = res

    pltpu.sync_copy(out_v, out_hbm.at[pl.ds(base, _BPW)])


def kernel(UserIdx, itemIdx, user_table, item_table):
    mesh = plsc.VectorSubcoreMesh(core_axis_name="core",
                                  subcore_axis_name="subcore")
    cp = pltpu.CompilerParams()
    if "needs_layout_passes" in pltpu.CompilerParams.__dataclass_fields__:
        cp = dataclasses.replace(cp, needs_layout_passes=False)
    mf = pl.kernel(
        _mf_body,
        out_type=jax.ShapeDtypeStruct((_B,), jnp.float32),
        mesh=mesh,
        scratch_types=[
            pltpu.VMEM((_NCHUNK, _C), jnp.int32),
            pltpu.VMEM((_NCHUNK, _C), jnp.int32),
            pltpu.VMEM((2, _C, _DIM), jnp.float32),
            pltpu.VMEM((2, _C, _DIM), jnp.float32),
            pltpu.VMEM((_BPW,), jnp.float32),
            pltpu.VMEM((_L * _PAD,), jnp.float32),
            pltpu.SemaphoreType.DMA,
            pltpu.SemaphoreType.DMA,
            pltpu.SemaphoreType.DMA,
            pltpu.SemaphoreType.DMA,
        ],
        compiler_params=cp,
    )
    return mf(UserIdx.astype(jnp.int32), itemIdx.astype(jnp.int32),
              user_table, item_table)
